# baseline (speedup 1.0000x reference)
import jax
import jax.numpy as jnp
from jax.experimental import pallas as pl
from jax.experimental.pallas import tpu as pltpu

B, H, W = 16, 384, 384


def _loss_kernel(label_ref, x_ref, p_ref, n_ref, out_ref, acc_ref, cnt_ref):
    b = pl.program_id(0)

    @pl.when(b == 0)
    def _init():
        acc_ref[0] = 0.0
        cnt_ref[0] = 0.0

    x = x_ref[0]
    p = p_ref[0]
    n = n_ref[0]
    m = jnp.logical_and(p > 0.5, n > 0.5).astype(jnp.float32)
    y = label_ref[b // 2].astype(jnp.float32)
    u = jnp.exp(-jnp.abs(x))
    t = 1.0 + u * m
    v = t
    for _ in range(5):
        half = v.shape[0] // 2
        v = v[:half] * v[half:]
    s_log = jnp.sum(jnp.log(v))
    s_max = jnp.sum(m * jnp.maximum(x, 0.0))
    s_x = jnp.sum(m * x)
    acc_ref[0] += (1.0 + y) * (s_max + s_log) - (2.0 * y) * s_x
    cnt_ref[0] += jnp.sum(m)

    @pl.when(b == pl.num_programs(0) - 1)
    def _fini():
        out_ref[0] = acc_ref[0] / cnt_ref[0]


def kernel(cancer_logits, prostate_mask, needle_mask, label, involvement):
    x = cancer_logits.reshape(B, H, W)
    p = prostate_mask.reshape(B, H, W)
    n = needle_mask.reshape(B, H, W)
    grid_spec = pltpu.PrefetchScalarGridSpec(
        num_scalar_prefetch=1,
        grid=(2 * B,),
        in_specs=[
            pl.BlockSpec((1, H // 2, W), lambda b, lbl: (b // 2, b % 2, 0)),
            pl.BlockSpec((1, H // 2, W), lambda b, lbl: (b // 2, b % 2, 0)),
            pl.BlockSpec((1, H // 2, W), lambda b, lbl: (b // 2, b % 2, 0)),
        ],
        out_specs=pl.BlockSpec(memory_space=pltpu.SMEM),
        scratch_shapes=[
            pltpu.SMEM((1,), jnp.float32),
            pltpu.SMEM((1,), jnp.float32),
        ],
    )
    out = pl.pallas_call(
        _loss_kernel,
        grid_spec=grid_spec,
        out_shape=jax.ShapeDtypeStruct((1,), jnp.float32),
    )(label.astype(jnp.int32), x, p, n)
    return out[0]


# speedup vs baseline: 1.8452x; 1.8452x over previous
import jax
import jax.numpy as jnp
from jax.experimental import pallas as pl
from jax.experimental.pallas import tpu as pltpu

B, H, W = 16, 384, 384
IMGS = 2


def _loss_kernel(label_ref, x_ref, p_ref, n_ref, out_ref, acc_ref, cnt_ref):
    g = pl.program_id(0)

    @pl.when(g == 0)
    def _init():
        acc_ref[0] = 0.0
        cnt_ref[0] = 0.0

    total = 0.0
    count = 0.0
    for j in range(IMGS):
        x = x_ref[j]
        p = p_ref[j]
        n = n_ref[j]
        m = jnp.logical_and(p > 0.5, n > 0.5).astype(jnp.float32)
        y = label_ref[g * IMGS + j].astype(jnp.float32)
        u = jnp.exp(-jnp.abs(x))
        t = 1.0 + u * m
        v = t
        for _ in range(6):
            half = v.shape[0] // 2
            v = v[:half] * v[half:]
        s_log = jnp.sum(jnp.log(v))
        s_max = jnp.sum(m * jnp.maximum(x, 0.0))
        s_x = jnp.sum(m * x)
        total += (1.0 + y) * (s_max + s_log) - (2.0 * y) * s_x
        count += jnp.sum(m)
    acc_ref[0] += total
    cnt_ref[0] += count

    @pl.when(g == pl.num_programs(0) - 1)
    def _fini():
        out_ref[0] = acc_ref[0] / cnt_ref[0]


def kernel(cancer_logits, prostate_mask, needle_mask, label, involvement):
    x = cancer_logits.reshape(B, H, W)
    p = prostate_mask.reshape(B, H, W)
    n = needle_mask.reshape(B, H, W)
    grid_spec = pltpu.PrefetchScalarGridSpec(
        num_scalar_prefetch=1,
        grid=(B // IMGS,),
        in_specs=[
            pl.BlockSpec((IMGS, H, W), lambda g, lbl: (g, 0, 0)),
            pl.BlockSpec((IMGS, H, W), lambda g, lbl: (g, 0, 0)),
            pl.BlockSpec((IMGS, H, W), lambda g, lbl: (g, 0, 0)),
        ],
        out_specs=pl.BlockSpec(memory_space=pltpu.SMEM),
        scratch_shapes=[
            pltpu.SMEM((1,), jnp.float32),
            pltpu.SMEM((1,), jnp.float32),
        ],
    )
    out = pl.pallas_call(
        _loss_kernel,
        grid_spec=grid_spec,
        out_shape=jax.ShapeDtypeStruct((1,), jnp.float32),
    )(label.astype(jnp.int32), x, p, n)
    return out[0]


# speedup vs baseline: 2.0737x; 1.1239x over previous
import jax
import jax.numpy as jnp
from jax.experimental import pallas as pl
from jax.experimental.pallas import tpu as pltpu

B, H, W = 16, 384, 384
IMGS = 4


def _loss_kernel(label_ref, x_ref, p_ref, n_ref, out_ref, acc_ref, cnt_ref):
    g = pl.program_id(0)

    @pl.when(g == 0)
    def _init():
        acc_ref[0] = 0.0
        cnt_ref[0] = 0.0

    total = 0.0
    count = 0.0
    for j in range(IMGS):
        x = x_ref[j]
        p = p_ref[j]
        n = n_ref[j]
        m = jnp.logical_and(p > 0.5, n > 0.5).astype(jnp.float32)
        y = label_ref[g * IMGS + j].astype(jnp.float32)
        u = jnp.exp(-jnp.abs(x))
        t = 1.0 + u * m
        v = t
        for _ in range(6):
            half = v.shape[0] // 2
            v = v[:half] * v[half:]
        s_log = jnp.sum(jnp.log(v))
        s_max = jnp.sum(m * jnp.maximum(x, 0.0))
        s_x = jnp.sum(m * x)
        total += (1.0 + y) * (s_max + s_log) - (2.0 * y) * s_x
        count += jnp.sum(m)
    acc_ref[0] += total
    cnt_ref[0] += count

    @pl.when(g == pl.num_programs(0) - 1)
    def _fini():
        out_ref[0] = acc_ref[0] / cnt_ref[0]


def kernel(cancer_logits, prostate_mask, needle_mask, label, involvement):
    x = cancer_logits.reshape(B, H, W)
    p = prostate_mask.reshape(B, H, W)
    n = needle_mask.reshape(B, H, W)
    grid_spec = pltpu.PrefetchScalarGridSpec(
        num_scalar_prefetch=1,
        grid=(B // IMGS,),
        in_specs=[
            pl.BlockSpec((IMGS, H, W), lambda g, lbl: (g, 0, 0)),
            pl.BlockSpec((IMGS, H, W), lambda g, lbl: (g, 0, 0)),
            pl.BlockSpec((IMGS, H, W), lambda g, lbl: (g, 0, 0)),
        ],
        out_specs=pl.BlockSpec(memory_space=pltpu.SMEM),
        scratch_shapes=[
            pltpu.SMEM((1,), jnp.float32),
            pltpu.SMEM((1,), jnp.float32),
        ],
    )
    out = pl.pallas_call(
        _loss_kernel,
        grid_spec=grid_spec,
        out_shape=jax.ShapeDtypeStruct((1,), jnp.float32),
    )(label.astype(jnp.int32), x, p, n)
    return out[0]
